# Initial kernel scaffold; baseline (speedup 1.0000x reference)
import jax
import jax.numpy as jnp
from jax.experimental import pallas as pl


def kernel(q, K, V, salience, topk):
    raise NotImplementedError("write your pallas kernel here")


# speedup vs baseline: 10.0781x; 10.0781x over previous
import functools
import math

import jax
import jax.numpy as jnp
from jax.experimental import pallas as pl
from jax.experimental.pallas import tpu as pltpu

_TBA = 128
_TBB = 256


def _select_body(q_ref, k_ref, sal_ref, wd_ref, aw_ref, *, n_top, inv_sqrt_d):
    scores = jax.lax.dot_general(
        q_ref[...], k_ref[...], (((1,), (1,)), ((), ())),
        preferred_element_type=jnp.float32)
    scores = scores * inv_sqrt_d + sal_ref[...]

    run = scores
    neg_inf = jnp.float32(-jnp.inf)
    vals = []
    for j in range(n_top):
        m = jnp.max(run, axis=1, keepdims=True)
        vals.append(m)
        if j != n_top - 1:
            run = jnp.where(run == m, neg_inf, run)
    vals = jnp.concatenate(vals, axis=1)

    m0 = vals[:, 0:1]
    e = jnp.exp(vals - m0)
    denom = jnp.sum(e, axis=1, keepdims=True)
    aw_ref[...] = e / denom

    thresh = vals[:, n_top - 1:n_top]
    wd_ref[...] = jnp.where(scores >= thresh,
                            jnp.exp(scores - m0) / denom,
                            jnp.float32(0.0))


def _combine_body(wd_ref, v_ref, rv_ref):
    rv_ref[...] = jax.lax.dot_general(
        wd_ref[...], v_ref[...], (((1,), (0,)), ((), ())),
        preferred_element_type=jnp.float32)


def kernel(q, K, V, salience, topk):
    Bq, Tq, Dq = q.shape
    S = K.shape[0]
    n_top = min(32, S)
    R = Bq * Tq
    q2 = q.reshape(R, Dq)
    sal2 = salience.reshape(1, S)

    sel = functools.partial(_select_body, n_top=n_top,
                            inv_sqrt_d=float(1.0 / math.sqrt(Dq)))
    wd, aw = pl.pallas_call(
        sel,
        grid=(R // _TBA,),
        in_specs=[
            pl.BlockSpec((_TBA, Dq), lambda i: (i, 0)),
            pl.BlockSpec((S, Dq), lambda i: (0, 0)),
            pl.BlockSpec((1, S), lambda i: (0, 0)),
        ],
        out_specs=[
            pl.BlockSpec((_TBA, S), lambda i: (i, 0)),
            pl.BlockSpec((_TBA, n_top), lambda i: (i, 0)),
        ],
        out_shape=[
            jax.ShapeDtypeStruct((R, S), jnp.float32),
            jax.ShapeDtypeStruct((R, n_top), jnp.float32),
        ],
        compiler_params=pltpu.CompilerParams(
            dimension_semantics=("arbitrary",),
        ),
    )(q2, K, sal2)

    rv = pl.pallas_call(
        _combine_body,
        grid=(R // _TBB,),
        in_specs=[
            pl.BlockSpec((_TBB, S), lambda i: (i, 0)),
            pl.BlockSpec((S, Dq), lambda i: (0, 0)),
        ],
        out_specs=pl.BlockSpec((_TBB, Dq), lambda i: (i, 0)),
        out_shape=jax.ShapeDtypeStruct((R, Dq), jnp.float32),
        compiler_params=pltpu.CompilerParams(
            dimension_semantics=("arbitrary",),
        ),
    )(wd, V)
    return rv.reshape(Bq, Tq, Dq), aw.reshape(Bq, Tq, n_top)


# speedup vs baseline: 10.2495x; 1.0170x over previous
import functools
import math

import jax
import jax.numpy as jnp
from jax.experimental import pallas as pl
from jax.experimental.pallas import tpu as pltpu

_TB = 128


def _body(q_ref, k_ref, v_ref, sal_ref, rv_ref, aw_ref, *, n_top, inv_sqrt_d):
    scores = jax.lax.dot_general(
        q_ref[...], k_ref[...], (((1,), (1,)), ((), ())),
        preferred_element_type=jnp.float32)
    scores = scores * inv_sqrt_d + sal_ref[...]

    neg_inf = jnp.float32(-jnp.inf)
    m = jnp.max(scores, axis=1, keepdims=True)
    vals = [m]
    for j in range(1, n_top):
        cand = jnp.where(scores < m, scores, neg_inf)
        m = jnp.max(cand, axis=1, keepdims=True)
        vals.append(m)
    vals = jnp.concatenate(vals, axis=1)

    m0 = vals[:, 0:1]
    e = jnp.exp(vals - m0)
    denom = jnp.sum(e, axis=1, keepdims=True)
    aw_ref[...] = e / denom

    thresh = vals[:, n_top - 1:n_top]
    wd = jnp.where(scores >= thresh,
                   jnp.exp(scores - m0) / denom,
                   jnp.float32(0.0)).astype(jnp.bfloat16)
    rv_ref[...] = jax.lax.dot_general(
        wd, v_ref[...], (((1,), (0,)), ((), ())),
        preferred_element_type=jnp.float32)


def kernel(q, K, V, salience, topk):
    Bq, Tq, Dq = q.shape
    S = K.shape[0]
    n_top = min(32, S)
    R = Bq * Tq
    q2 = q.reshape(R, Dq)
    sal2 = salience.reshape(1, S)
    v16 = V.astype(jnp.bfloat16)

    body = functools.partial(_body, n_top=n_top,
                             inv_sqrt_d=float(1.0 / math.sqrt(Dq)))
    rv, aw = pl.pallas_call(
        body,
        grid=(R // _TB,),
        in_specs=[
            pl.BlockSpec((_TB, Dq), lambda i: (i, 0)),
            pl.BlockSpec((S, Dq), lambda i: (0, 0)),
            pl.BlockSpec((S, Dq), lambda i: (0, 0)),
            pl.BlockSpec((1, S), lambda i: (0, 0)),
        ],
        out_specs=[
            pl.BlockSpec((_TB, Dq), lambda i: (i, 0)),
            pl.BlockSpec((_TB, n_top), lambda i: (i, 0)),
        ],
        out_shape=[
            jax.ShapeDtypeStruct((R, Dq), jnp.float32),
            jax.ShapeDtypeStruct((R, n_top), jnp.float32),
        ],
        compiler_params=pltpu.CompilerParams(
            dimension_semantics=("arbitrary",),
        ),
    )(q2, K, v16, sal2)
    return rv.reshape(Bq, Tq, Dq), aw.reshape(Bq, Tq, n_top)
